# baseline (speedup 1.0000x reference)
import functools

import jax
import jax.numpy as jnp
from jax.experimental import pallas as pl
from jax.experimental.pallas import tpu as pltpu


def _fused_body(n_rows, g, r, x_ref, w_ref, gamma_ref, beta_ref,
                out_ref, h_any, stats, hbuf, ibuf, sem_out, sem_in):
    i = pl.program_id(0)

    @pl.when(i < g)
    def _phase0():
        h = jax.lax.dot_general(
            x_ref[...].astype(jnp.bfloat16), w_ref[...].astype(jnp.bfloat16),
            dimension_numbers=(((1,), (1,)), ((), ())),
            preferred_element_type=jnp.float32,
        )
        slot = jax.lax.rem(i, 2)

        @pl.when(i >= 2)
        def _():
            pltpu.make_async_copy(
                hbuf.at[slot], h_any.at[pl.ds(jnp.maximum(i - 2, 0) * r, r)],
                sem_out.at[slot]
            ).wait()

        hbuf[slot] = h.astype(jnp.bfloat16)
        pltpu.make_async_copy(
            hbuf.at[slot], h_any.at[pl.ds(jnp.minimum(i, g - 1) * r, r)],
            sem_out.at[slot]
        ).start()

        s = jnp.sum(h, axis=0)
        ss = jnp.sum(h * h, axis=0)
        row = jax.lax.broadcasted_iota(jnp.int32, stats.shape, 0)
        contrib = (jnp.where(row == 0, s[None, :], 0.0)
                   + jnp.where(row == 1, ss[None, :], 0.0))

        @pl.when(i == 0)
        def _():
            stats[...] = contrib

        @pl.when(i != 0)
        def _():
            stats[...] += contrib

        @pl.when(i == g - 1)
        def _():
            if g <= 2:
                pltpu.make_async_copy(
                    hbuf.at[0], h_any.at[pl.ds(0, r)], sem_out.at[0]
                ).wait()
            pltpu.make_async_copy(
                h_any.at[pl.ds(0, r)], ibuf.at[0], sem_in.at[0]
            ).start()

    @pl.when(i >= g)
    def _phase1():
        j = jnp.clip(i - g, 0, g - 1)
        slot = jax.lax.rem(j, 2)

        _drain = [k for k in range(max(g - 2, 0), g) if not (g <= 2 and k == 0)]
        if _drain:
            @pl.when(j == 0)
            def _():
                for k in _drain:
                    pltpu.make_async_copy(
                        hbuf.at[k % 2],
                        h_any.at[pl.ds(k * r, r)],
                        sem_out.at[k % 2],
                    ).wait()

        @pl.when(j + 1 < g)
        def _():
            jn = jnp.minimum(j + 1, g - 1)
            nslot = jax.lax.rem(jn, 2)
            pltpu.make_async_copy(
                h_any.at[pl.ds(jn * r, r)], ibuf.at[nslot], sem_in.at[nslot]
            ).start()

        pltpu.make_async_copy(
            h_any.at[pl.ds(j * r, r)], ibuf.at[slot], sem_in.at[slot]
        ).wait()

        st = stats[...]
        mean = st[0:1, :] / n_rows
        ex2 = st[1:2, :] / n_rows
        var = ex2 - mean * mean
        inv = jax.lax.rsqrt(var + 1e-5)
        scale = gamma_ref[...] * inv
        bias = beta_ref[...] - mean * scale
        out_ref[...] = jnp.maximum(
            ibuf[slot].astype(jnp.float32) * scale + bias, 0.0)


def _pick_tile(n, candidates):
    for c in candidates:
        if n % c == 0 and c % 8 == 0:
            return c
    return n


def kernel(p, x, o, W, gamma, beta):
    n, c_in = x.shape
    c_out = W.shape[0]

    r = _pick_tile(n, (2000, 1000, 8))
    g = n // r
    out, _ = pl.pallas_call(
        functools.partial(_fused_body, float(n), g, r),
        grid=(2 * g,),
        in_specs=[
            pl.BlockSpec((r, c_in), lambda i, g=g: (jnp.where(i < g, i, g - 1), 0)),
            pl.BlockSpec((c_out, c_in), lambda i: (0, 0)),
            pl.BlockSpec((1, c_out), lambda i: (0, 0)),
            pl.BlockSpec((1, c_out), lambda i: (0, 0)),
        ],
        out_specs=[
            pl.BlockSpec((r, c_out), lambda i, g=g: (jnp.where(i < g, 0, i - g), 0)),
            pl.BlockSpec(memory_space=pltpu.MemorySpace.HBM),
        ],
        out_shape=[
            jax.ShapeDtypeStruct((n, c_out), jnp.float32),
            jax.ShapeDtypeStruct((n, c_out), jnp.bfloat16),
        ],
        scratch_shapes=[
            pltpu.VMEM((8, c_out), jnp.float32),
            pltpu.VMEM((2, r, c_out), jnp.bfloat16),
            pltpu.VMEM((2, r, c_out), jnp.bfloat16),
            pltpu.SemaphoreType.DMA((2,)),
            pltpu.SemaphoreType.DMA((2,)),
        ],
    )(x, W, gamma.reshape(1, c_out), beta.reshape(1, c_out))

    return (p, out, o)


# speedup vs baseline: 1.1104x; 1.1104x over previous
import functools

import jax
import jax.numpy as jnp
from jax.experimental import pallas as pl
from jax.experimental.pallas import tpu as pltpu


def _normalize(h_bf16, n_rows, stats, gamma_ref, beta_ref):
    st = stats[...]
    mean = st[0:1, :] / n_rows
    ex2 = st[1:2, :] / n_rows
    var = ex2 - mean * mean
    inv = jax.lax.rsqrt(var + 1e-5)
    scale = gamma_ref[...] * inv
    bias = beta_ref[...] - mean * scale
    return jnp.maximum(h_bf16.astype(jnp.float32) * scale + bias, 0.0)


def _fused_body(n_rows, g, r, x_ref, w_ref, gamma_ref, beta_ref,
                out_ref, h_any, stats, hbuf, ibuf, sem_out, sem_in):
    i = pl.program_id(0)
    res = min(g, 2)

    @pl.when(i < g)
    def _phase0():
        h = jax.lax.dot_general(
            x_ref[...].astype(jnp.bfloat16), w_ref[...],
            dimension_numbers=(((1,), (1,)), ((), ())),
            preferred_element_type=jnp.float32,
        )
        slot = jax.lax.rem(i, 2)

        @pl.when(i >= 2)
        def _():
            pltpu.make_async_copy(
                hbuf.at[slot], h_any.at[pl.ds(jnp.maximum(i - 2, 0) * r, r)],
                sem_out.at[slot]
            ).wait()

        hbuf[slot] = h.astype(jnp.bfloat16)

        @pl.when(i < g - res)
        def _():
            pltpu.make_async_copy(
                hbuf.at[slot], h_any.at[pl.ds(jnp.minimum(i, g - 1) * r, r)],
                sem_out.at[slot]
            ).start()

        s = jnp.sum(h, axis=0)
        ss = jnp.sum(h * h, axis=0)
        row = jax.lax.broadcasted_iota(jnp.int32, stats.shape, 0)
        contrib = (jnp.where(row == 0, s[None, :], 0.0)
                   + jnp.where(row == 1, ss[None, :], 0.0))

        @pl.when(i == 0)
        def _():
            stats[...] = contrib

        @pl.when(i != 0)
        def _():
            stats[...] += contrib

        if g > res:
            @pl.when(i == g - 1)
            def _():
                pltpu.make_async_copy(
                    h_any.at[pl.ds(0, r)], ibuf.at[0], sem_in.at[0]
                ).start()

    @pl.when(i >= g)
    def _phase1():
        j = jnp.clip(i - g, 0, g - 1)

        if g > res + 1:
            @pl.when(jnp.logical_and(j + 1 - res >= 1,
                                     j + 1 - res <= g - res - 1))
            def _():
                u = jnp.clip(j + 1 - res, 1, g - res - 1)
                nslot = jax.lax.rem(u, 2)
                pltpu.make_async_copy(
                    h_any.at[pl.ds(u * r, r)], ibuf.at[nslot], sem_in.at[nslot]
                ).start()

        @pl.when(j < res)
        def _():
            hslot = jax.lax.rem(g - res + j, 2)
            out_ref[...] = _normalize(hbuf[hslot], n_rows, stats,
                                      gamma_ref, beta_ref)

        if g > res:
            @pl.when(j >= res)
            def _():
                u = jnp.clip(j - res, 0, g - res - 1)
                islot = jax.lax.rem(u, 2)
                pltpu.make_async_copy(
                    h_any.at[pl.ds(u * r, r)], ibuf.at[islot], sem_in.at[islot]
                ).wait()
                out_ref[...] = _normalize(ibuf[islot], n_rows, stats,
                                          gamma_ref, beta_ref)


def _pick_tile(n, candidates):
    for c in candidates:
        if n % c == 0 and c % 8 == 0:
            return c
    return n


def kernel(p, x, o, W, gamma, beta):
    n, c_in = x.shape
    c_out = W.shape[0]

    r = _pick_tile(n, (4000, 2000, 1000, 8))
    g = n // r
    res = min(g, 2)

    def _out_idx(i, g=g, res=res):
        j = jnp.clip(i - g, 0, g - 1)
        t = jnp.where(j < res, g - res + j, j - res)
        return (jnp.where(i < g, g - res, t), 0)

    out, _ = pl.pallas_call(
        functools.partial(_fused_body, float(n), g, r),
        grid=(2 * g,),
        in_specs=[
            pl.BlockSpec((r, c_in), lambda i, g=g: (jnp.where(i < g, i, g - 1), 0)),
            pl.BlockSpec((c_out, c_in), lambda i: (0, 0)),
            pl.BlockSpec((1, c_out), lambda i: (0, 0)),
            pl.BlockSpec((1, c_out), lambda i: (0, 0)),
        ],
        out_specs=[
            pl.BlockSpec((r, c_out), _out_idx),
            pl.BlockSpec(memory_space=pltpu.MemorySpace.HBM),
        ],
        out_shape=[
            jax.ShapeDtypeStruct((n, c_out), jnp.float32),
            jax.ShapeDtypeStruct((n, c_out), jnp.bfloat16),
        ],
        scratch_shapes=[
            pltpu.VMEM((8, c_out), jnp.float32),
            pltpu.VMEM((2, r, c_out), jnp.bfloat16),
            pltpu.VMEM((2, r, c_out), jnp.bfloat16),
            pltpu.SemaphoreType.DMA((2,)),
            pltpu.SemaphoreType.DMA((2,)),
        ],
    )(x, W.astype(jnp.bfloat16), gamma.reshape(1, c_out),
      beta.reshape(1, c_out))

    return (p, out, o)


# speedup vs baseline: 1.1205x; 1.0091x over previous
import functools

import jax
import jax.numpy as jnp
from jax.experimental import pallas as pl
from jax.experimental.pallas import tpu as pltpu


def _normalize(h_bf16, n_rows, stats, gamma_ref, beta_ref):
    st = stats[...]
    mean = st[0:1, :] / n_rows
    ex2 = st[1:2, :] / n_rows
    var = ex2 - mean * mean
    inv = jax.lax.rsqrt(var + 1e-5)
    scale = gamma_ref[...] * inv
    bias = beta_ref[...] - mean * scale
    return jnp.maximum(h_bf16.astype(jnp.float32) * scale + bias, 0.0)


def _fused_body(n_rows, g, r, x_ref, w_ref, gamma_ref, beta_ref,
                out_ref, h_any, stats, hbuf, ibuf, sem_out, sem_in):
    i = pl.program_id(0)
    res = min(g, 2)

    @pl.when(i < g)
    def _phase0():
        h = jax.lax.dot_general(
            x_ref[...].astype(jnp.bfloat16), w_ref[...].astype(jnp.bfloat16),
            dimension_numbers=(((1,), (1,)), ((), ())),
            preferred_element_type=jnp.float32,
        )
        slot = jax.lax.rem(i, 2)

        @pl.when(i >= 2)
        def _():
            pltpu.make_async_copy(
                hbuf.at[slot], h_any.at[pl.ds(jnp.maximum(i - 2, 0) * r, r)],
                sem_out.at[slot]
            ).wait()

        hbuf[slot] = h.astype(jnp.bfloat16)

        @pl.when(i < g - res)
        def _():
            pltpu.make_async_copy(
                hbuf.at[slot], h_any.at[pl.ds(jnp.minimum(i, g - 1) * r, r)],
                sem_out.at[slot]
            ).start()

        s = jnp.sum(h, axis=0)
        ss = jnp.sum(h * h, axis=0)
        row = jax.lax.broadcasted_iota(jnp.int32, stats.shape, 0)
        contrib = (jnp.where(row == 0, s[None, :], 0.0)
                   + jnp.where(row == 1, ss[None, :], 0.0))

        @pl.when(i == 0)
        def _():
            stats[...] = contrib

        @pl.when(i != 0)
        def _():
            stats[...] += contrib

        if g > res:
            @pl.when(i == g - 1)
            def _():
                pltpu.make_async_copy(
                    h_any.at[pl.ds(0, r)], ibuf.at[0], sem_in.at[0]
                ).start()

    @pl.when(i >= g)
    def _phase1():
        j = jnp.clip(i - g, 0, g - 1)

        if g > res + 1:
            @pl.when(jnp.logical_and(j + 1 - res >= 1,
                                     j + 1 - res <= g - res - 1))
            def _():
                u = jnp.clip(j + 1 - res, 1, g - res - 1)
                nslot = jax.lax.rem(u, 2)
                pltpu.make_async_copy(
                    h_any.at[pl.ds(u * r, r)], ibuf.at[nslot], sem_in.at[nslot]
                ).start()

        @pl.when(j < res)
        def _():
            hslot = jax.lax.rem(g - res + j, 2)
            out_ref[...] = _normalize(hbuf[hslot], n_rows, stats,
                                      gamma_ref, beta_ref)

        if g > res:
            @pl.when(j >= res)
            def _():
                u = jnp.clip(j - res, 0, g - res - 1)
                islot = jax.lax.rem(u, 2)
                pltpu.make_async_copy(
                    h_any.at[pl.ds(u * r, r)], ibuf.at[islot], sem_in.at[islot]
                ).wait()
                out_ref[...] = _normalize(ibuf[islot], n_rows, stats,
                                          gamma_ref, beta_ref)


def _pick_tile(n, candidates):
    for c in candidates:
        if n % c == 0 and c % 8 == 0:
            return c
    return n


def kernel(p, x, o, W, gamma, beta):
    n, c_in = x.shape
    c_out = W.shape[0]

    r = _pick_tile(n, (4000, 2000, 1000, 8))
    g = n // r
    res = min(g, 2)

    def _out_idx(i, g=g, res=res):
        j = jnp.clip(i - g, 0, g - 1)
        t = jnp.where(j < res, g - res + j, j - res)
        return (jnp.where(i < g, g - res, t), 0)

    out, _ = pl.pallas_call(
        functools.partial(_fused_body, float(n), g, r),
        grid=(2 * g,),
        in_specs=[
            pl.BlockSpec((r, c_in), lambda i, g=g: (jnp.where(i < g, i, g - 1), 0)),
            pl.BlockSpec((c_out, c_in), lambda i: (0, 0)),
            pl.BlockSpec((1, c_out), lambda i: (0, 0)),
            pl.BlockSpec((1, c_out), lambda i: (0, 0)),
        ],
        out_specs=[
            pl.BlockSpec((r, c_out), _out_idx),
            pl.BlockSpec(memory_space=pltpu.MemorySpace.HBM),
        ],
        out_shape=[
            jax.ShapeDtypeStruct((n, c_out), jnp.float32),
            jax.ShapeDtypeStruct((n, c_out), jnp.bfloat16),
        ],
        scratch_shapes=[
            pltpu.VMEM((8, c_out), jnp.float32),
            pltpu.VMEM((2, r, c_out), jnp.bfloat16),
            pltpu.VMEM((2, r, c_out), jnp.bfloat16),
            pltpu.SemaphoreType.DMA((2,)),
            pltpu.SemaphoreType.DMA((2,)),
        ],
    )(x, W, gamma.reshape(1, c_out), beta.reshape(1, c_out))

    return (p, out, o)
